# baseline (speedup 1.0000x reference)
import functools
import jax
import jax.numpy as jnp
from jax import lax
from jax.experimental import pallas as pl
from jax.experimental.pallas import tpu as pltpu
from jax.experimental.pallas import tpu_sc as plsc

_EPS = 1e-6
_M = 16
_L = 16
_NC = 2


def _a_body(q_ref, c_ref, sim_ref):
    c = c_ref[0]
    cn = c / (jnp.sqrt(jnp.sum(c * c, axis=1, keepdims=True)) + _EPS)
    q = q_ref[0]
    qn = q / (jnp.sqrt(jnp.sum(q * q, axis=1, keepdims=True)) + _EPS)
    w_t = lax.dot_general(cn, qn, (((1,), (1,)), ((), ())),
                          preferred_element_type=jnp.float32)
    sim_ref[0] = jnp.max(w_t, axis=0, keepdims=True)


def _sim_stage(Q_neg, C_rows, split=4):
    B, K, D = Q_neg.shape
    R = C_rows.shape[1]
    ks = K // split
    q_view = jnp.reshape(Q_neg, (B * split, ks, D))
    sim = pl.pallas_call(
        _a_body,
        grid=(B * split,),
        in_specs=[
            pl.BlockSpec((1, ks, D), lambda b: (b, 0, 0)),
            pl.BlockSpec((1, R, D), lambda b: (b // split, 0, 0)),
        ],
        out_specs=pl.BlockSpec((1, 1, ks), lambda b: (b, 0, 0)),
        out_shape=jax.ShapeDtypeStruct((B * split, 1, ks), jnp.float32),
    )(q_view, C_rows)
    return jnp.reshape(sim, (B, K))


def _sc_topk_gather(sim, q_flat, B, K, D):
    n_chunks = K // _L
    mesh = plsc.VectorSubcoreMesh(core_axis_name="c", subcore_axis_name="s")

    @functools.partial(
        pl.kernel,
        out_type=[
            jax.ShapeDtypeStruct((B, _M, D), jnp.float32),
            jax.ShapeDtypeStruct((B, _L), jnp.float32),
        ],
        mesh=mesh,
        scratch_types=[
            pltpu.VMEM((K,), jnp.float32),
            pltpu.VMEM((K,), jnp.float32),
            pltpu.VMEM((K + _L,), jnp.float32),
            pltpu.VMEM((K + _L,), jnp.int32),
            pltpu.VMEM((_M, D), jnp.float32),
            pltpu.VMEM((_L,), jnp.float32),
            pltpu.SemaphoreType.DMA,
        ],
        compiler_params=pltpu.CompilerParams(needs_layout_passes=False),
    )
    def sc_kernel(sim_hbm, q_hbm, qsel_hbm, sums_hbm,
                  simv, exclv, candv, candi, rowsv, sumv, sem):
        b = lax.axis_index("s") * _NC + lax.axis_index("c")
        lane = lax.iota(jnp.int32, _L)
        neg_inf = jnp.full((_L,), -jnp.inf, jnp.float32)
        big_i = jnp.full((_L,), 2**30, jnp.int32)

        pltpu.sync_copy(sim_hbm.at[b], simv)

        def loop1(i, carry):
            bmax, ssum = carry
            s = simv[pl.ds(i * _L, _L)]
            e = 1.0 - s
            exclv[pl.ds(i * _L, _L)] = e
            return jnp.maximum(bmax, e), ssum + s

        bmax, ssum = lax.fori_loop(
            0, n_chunks, loop1,
            (neg_inf, jnp.zeros((_L,), jnp.float32)))
        t = jnp.min(bmax)
        sumv[...] = jnp.broadcast_to(jnp.sum(ssum), (_L,))
        pltpu.sync_copy(sumv, sums_hbm.at[b])

        def loop2(i, n):
            e = exclv[pl.ds(i * _L, _L)]
            m = e >= t
            plsc.store_compressed(candv.at[pl.ds(n, _L)], e, mask=m)
            plsc.store_compressed(candi.at[pl.ds(n, _L)], lane + i * _L, mask=m)
            cnt = plsc.all_reduce_population_count(m)
            return n + jnp.max(cnt)

        n = lax.fori_loop(0, n_chunks, loop2, jnp.int32(0))
        candv[pl.ds(n, _L)] = neg_inf
        candi[pl.ds(n, _L)] = jnp.full((_L,), K, jnp.int32)
        nch = (n + _L - 1) // _L

        sel = jnp.zeros((_L,), jnp.int32)
        for j in range(_M):
            def sloop(ci, carry):
                bv, bi, bp = carry
                v = candv[pl.ds(ci * _L, _L)]
                ii = candi[pl.ds(ci * _L, _L)]
                pp = lane + ci * _L
                better = (v > bv) | ((v == bv) & (ii < bi))
                return (jnp.where(better, v, bv),
                        jnp.where(better, ii, bi),
                        jnp.where(better, pp, bp))

            bv, bi, bp = lax.fori_loop(0, nch, sloop,
                                       (neg_inf, big_i, big_i))
            gv = jnp.max(bv)
            eq = bv == gv
            gidx = jnp.min(jnp.where(eq, bi, big_i))
            gpos = jnp.min(jnp.where(eq & (bi == gidx), bp, big_i))
            sel = jnp.where(lane == j, gidx, sel)
            plsc.store_scatter(candv, [jnp.broadcast_to(gpos, (_L,))],
                               neg_inf, mask=lane == 0)

        gids = sel + b * K
        pltpu.async_copy(q_hbm.at[gids], rowsv, sem).wait()
        pltpu.sync_copy(rowsv, qsel_hbm.at[b])

    return sc_kernel(sim, q_flat)


def _d_body(qsel_ref, c_ref, sums_ref, out_ref, mean_ref, *, B, K, R, D):
    call = jnp.reshape(c_ref[...], (B * R, D))
    cn = call / (jnp.sqrt(jnp.sum(call * call, axis=1, keepdims=True)) + _EPS)
    qs = jnp.reshape(qsel_ref[...], (B * _M, D))
    qsn = qs / (jnp.sqrt(jnp.sum(qs * qs, axis=1, keepdims=True)) + _EPS)
    for b in range(B):
        cnb = cn[b * R:(b + 1) * R]
        qb = qsn[b * _M:(b + 1) * _M]
        w = lax.dot_general(qb, cnb, (((1,), (1,)), ((), ())),
                            preferred_element_type=jnp.float32)
        p = lax.dot_general(w, cnb, (((1,), (0,)), ((), ())),
                            preferred_element_type=jnp.float32)
        r = qb - p
        out_ref[b] = r / (jnp.sqrt(jnp.sum(r * r, axis=1, keepdims=True))
                          + _EPS)
    s = sums_ref[...]
    mean_ref[...] = jnp.reshape(jnp.sum(s[:, 0:1]) / (B * K), (1, 1))


def kernel(Q_neg, C_rows):
    B, K, D = Q_neg.shape
    R = C_rows.shape[1]
    sim = _sim_stage(Q_neg, C_rows)
    q_flat = jnp.reshape(Q_neg, (B * K, D))
    qsel, sums = _sc_topk_gather(sim, q_flat, B, K, D)
    neg_refs, mean = pl.pallas_call(
        functools.partial(_d_body, B=B, K=K, R=R, D=D),
        out_shape=[
            jax.ShapeDtypeStruct((B, _M, D), jnp.float32),
            jax.ShapeDtypeStruct((1, 1), jnp.float32),
        ],
    )(qsel, C_rows, sums)
    return (neg_refs, mean[0, 0])


# speedup vs baseline: 1.6714x; 1.6714x over previous
import functools
import jax
import jax.numpy as jnp
from jax import lax
from jax.experimental import pallas as pl
from jax.experimental.pallas import tpu as pltpu
from jax.experimental.pallas import tpu_sc as plsc

_EPS = 1e-6
_M = 16
_L = 16
_NC = 2


def _a_body(q_ref, c_ref, sim_ref):
    c = c_ref[0]
    cn = c / (jnp.sqrt(jnp.sum(c * c, axis=1, keepdims=True)) + _EPS)
    q = q_ref[0]
    qn = q / (jnp.sqrt(jnp.sum(q * q, axis=1, keepdims=True)) + _EPS)
    w_t = lax.dot_general(cn, qn, (((1,), (1,)), ((), ())),
                          preferred_element_type=jnp.float32)
    sim_ref[0] = jnp.max(w_t, axis=0, keepdims=True)


def _sim_stage(Q_neg, C_rows):
    B, K, D = Q_neg.shape
    R = C_rows.shape[1]
    sim = pl.pallas_call(
        _a_body,
        grid=(B,),
        in_specs=[
            pl.BlockSpec((1, K, D), lambda b: (b, 0, 0)),
            pl.BlockSpec((1, R, D), lambda b: (b, 0, 0)),
        ],
        out_specs=pl.BlockSpec((1, 1, K), lambda b: (b, 0, 0)),
        out_shape=jax.ShapeDtypeStruct((B, 1, K), jnp.float32),
    )(Q_neg, C_rows)
    return jnp.reshape(sim, (B, K))


def _sc_topk_gather(sim, q_flat, B, K, D):
    n_chunks = K // _L
    mesh = plsc.VectorSubcoreMesh(core_axis_name="c", subcore_axis_name="s")

    @functools.partial(
        pl.kernel,
        out_type=[
            jax.ShapeDtypeStruct((B, _M, D), jnp.float32),
            jax.ShapeDtypeStruct((B, _L), jnp.float32),
        ],
        mesh=mesh,
        scratch_types=[
            pltpu.VMEM((K,), jnp.float32),
            pltpu.VMEM((K + _L,), jnp.float32),
            pltpu.VMEM((K + _L,), jnp.int32),
            pltpu.VMEM((_M, D), jnp.float32),
            pltpu.VMEM((_L,), jnp.float32),
            pltpu.SemaphoreType.DMA,
        ],
        compiler_params=pltpu.CompilerParams(needs_layout_passes=False),
    )
    def sc_kernel(sim_hbm, q_hbm, qsel_hbm, sums_hbm,
                  simv, exclv, candi, rowsv, sumv, sem):
        b = lax.axis_index("s") * _NC + lax.axis_index("c")
        lane = lax.iota(jnp.int32, _L)
        neg_inf = jnp.full((_L,), -jnp.inf, jnp.float32)
        big_i = jnp.full((_L,), 2**30, jnp.int32)

        pltpu.sync_copy(sim_hbm.at[b], simv)

        @plsc.parallel_loop(0, n_chunks, 4, unroll=4,
                            carry=(neg_inf, neg_inf,
                                   jnp.zeros((_L,), jnp.float32)))
        def p1(i, carry):
            bmax0, bmax1, ssum = carry
            for u in range(4):
                s = simv[pl.ds((i + u) * _L, _L)]
                e = 1.0 - s
                exclv[pl.ds((i + u) * _L, _L)] = e
                if u % 2 == 0:
                    bmax0 = jnp.maximum(bmax0, e)
                else:
                    bmax1 = jnp.maximum(bmax1, e)
                ssum = ssum + s
            return bmax0, bmax1, ssum

        bmax0, bmax1, ssum = p1
        t = jnp.min(jnp.maximum(bmax0, bmax1))
        sumv[...] = jnp.broadcast_to(jnp.sum(ssum), (_L,))
        pltpu.sync_copy(sumv, sums_hbm.at[b])

        def loop2(i, n):
            offs = n
            for u in range(4):
                m = exclv[pl.ds((i * 4 + u) * _L, _L)] >= t
                plsc.store_compressed(candi.at[pl.ds(offs, _L)],
                                      lane + (i * 4 + u) * _L, mask=m)
                cnt = plsc.all_reduce_population_count(m)
                offs = offs + jnp.max(cnt)
            return offs

        n = lax.fori_loop(0, n_chunks // 4, loop2, jnp.int32(0))
        candi[pl.ds(n, _L)] = jnp.full((_L,), K, jnp.int32)
        exclv[pl.ds(K, _L)] = neg_inf
        nch = (n + _L - 1) // _L

        sel = jnp.zeros((_L,), jnp.int32)
        for j in range(_M):
            def sloop(ci, carry):
                bv, bi = carry
                ii = candi[pl.ds(ci * _L, _L)]
                v = plsc.load_gather(exclv, [ii])
                better = (v > bv) | ((v == bv) & (ii < bi))
                return (jnp.where(better, v, bv),
                        jnp.where(better, ii, bi))

            bv, bi = lax.fori_loop(0, nch, sloop, (neg_inf, big_i))
            gv = jnp.max(bv)
            gidx = jnp.min(jnp.where(bv == gv, bi, big_i))
            sel = jnp.where(lane == j, gidx, sel)
            plsc.store_scatter(exclv, [jnp.broadcast_to(gidx, (_L,))],
                               neg_inf, mask=lane == 0)

        gids = sel + b * K
        pltpu.async_copy(q_hbm.at[gids], rowsv, sem).wait()
        pltpu.sync_copy(rowsv, qsel_hbm.at[b])

    return sc_kernel(sim, q_flat)


def _d_body(qsel_ref, c_ref, sums_ref, out_ref, mean_ref, *, B, K, R, D):
    call = jnp.reshape(c_ref[...], (B * R, D))
    cn = call / (jnp.sqrt(jnp.sum(call * call, axis=1, keepdims=True)) + _EPS)
    qs = jnp.reshape(qsel_ref[...], (B * _M, D))
    qsn = qs / (jnp.sqrt(jnp.sum(qs * qs, axis=1, keepdims=True)) + _EPS)
    for b in range(B):
        cnb = cn[b * R:(b + 1) * R]
        qb = qsn[b * _M:(b + 1) * _M]
        w = lax.dot_general(qb, cnb, (((1,), (1,)), ((), ())),
                            preferred_element_type=jnp.float32)
        p = lax.dot_general(w, cnb, (((1,), (0,)), ((), ())),
                            preferred_element_type=jnp.float32)
        r = qb - p
        out_ref[b] = r / (jnp.sqrt(jnp.sum(r * r, axis=1, keepdims=True))
                          + _EPS)
    s = sums_ref[...]
    mean_ref[...] = jnp.reshape(jnp.sum(s[:, 0:1]) / (B * K), (1, 1))


def kernel(Q_neg, C_rows):
    B, K, D = Q_neg.shape
    R = C_rows.shape[1]
    sim = _sim_stage(Q_neg, C_rows)
    q_flat = jnp.reshape(Q_neg, (B * K, D))
    qsel, sums = _sc_topk_gather(sim, q_flat, B, K, D)
    neg_refs, mean = pl.pallas_call(
        functools.partial(_d_body, B=B, K=K, R=R, D=D),
        out_shape=[
            jax.ShapeDtypeStruct((B, _M, D), jnp.float32),
            jax.ShapeDtypeStruct((1, 1), jnp.float32),
        ],
    )(qsel, C_rows, sums)
    return (neg_refs, mean[0, 0])
